# baseline (speedup 1.0000x reference)
import functools

import jax
import jax.numpy as jnp
from jax import lax
from jax.experimental import pallas as pl
from jax.experimental.pallas import tpu as pltpu
from jax.experimental.pallas import tpu_sc as plsc


def _gather_call(n_total, d, nw, nc, g, idx3, weight):
    per_w = n_total // nw
    nchunk = per_w // g
    mesh = plsc.VectorSubcoreMesh(core_axis_name="c", subcore_axis_name="s")

    @functools.partial(
        pl.kernel,
        mesh=mesh,
        out_type=jax.ShapeDtypeStruct((n_total, d), jnp.float32),
        compiler_params=pltpu.CompilerParams(use_tc_tiling_on_sc=False),
        scratch_types=[
            pltpu.VMEM((nchunk, g), jnp.int32),
            pltpu.VMEM((g, d), jnp.float32),
            pltpu.SemaphoreType.DMA,
        ],
    )
    def k(idx_hbm, table_hbm, out_hbm, idx_v, rows_v, sem):
        wid = lax.axis_index("s") * nc + lax.axis_index("c")
        pltpu.sync_copy(idx_hbm.at[wid], idx_v)
        base = wid * per_w

        def body(j, carry):
            pltpu.async_copy(table_hbm.at[idx_v.at[j]], rows_v, sem).wait()
            pltpu.sync_copy(rows_v, out_hbm.at[pl.ds(base + j * g, g)])
            return carry

        lax.fori_loop(0, nchunk, body, 0)

    return k(idx3, weight)


def kernel(token_ids, weight):
    b, s = token_ids.shape
    v, d = weight.shape
    n_total = b * s
    info = plsc.get_sparse_core_info()
    nc, ns = info.num_cores, info.num_subcores
    nw = nc * ns
    g = 128
    idx3 = token_ids.reshape(nw, n_total // (nw * g), g)
    out = _gather_call(n_total, d, nw, nc, g, idx3, weight)
    return out.reshape(b, s, d)


# speedup vs baseline: 1.1182x; 1.1182x over previous
import functools

import jax
import jax.numpy as jnp
from jax import lax
from jax.experimental import pallas as pl
from jax.experimental.pallas import tpu as pltpu
from jax.experimental.pallas import tpu_sc as plsc


def _gather_call(n_total, d, nw, nc, g, nbuf, h, idx3, weight):
    per_w = n_total // nw
    nchunk = per_w // g
    n_outer = nchunk // nbuf
    mesh = plsc.VectorSubcoreMesh(core_axis_name="c", subcore_axis_name="s")

    scratch = [
        pltpu.VMEM((nchunk, g), jnp.int32),
        pltpu.VMEM((nbuf, g, d), jnp.float32),
    ] + [pltpu.SemaphoreType.DMA] * (2 * nbuf)

    @functools.partial(
        pl.kernel,
        mesh=mesh,
        out_type=jax.ShapeDtypeStruct((n_total, d), jnp.float32),
        compiler_params=pltpu.CompilerParams(use_tc_tiling_on_sc=False),
        scratch_types=scratch,
    )
    def k(idx_hbm, table_hbm, out_hbm, idx_v, rows_v, *sems):
        gsem = sems[:nbuf]
        osem = sems[nbuf:]
        wid = lax.axis_index("s") * nc + lax.axis_index("c")
        pltpu.sync_copy(idx_hbm.at[wid], idx_v)
        base = wid * per_w

        def gather_desc(c, slot):
            return pltpu.make_async_copy(
                table_hbm.at[idx_v.at[c]], rows_v.at[slot], gsem[slot]
            )

        def out_desc(c, slot):
            return pltpu.make_async_copy(
                rows_v.at[slot], out_hbm.at[pl.ds(base + c * g, g)], osem[slot]
            )

        for b in range(h):
            gather_desc(b, b).start()

        def body(outer, carry):
            for b in range(nbuf):
                j = outer * nbuf + b
                gather_desc(j, b).wait()
                out_desc(j, b).start()
                jg = j + h
                b2 = (b + h) % nbuf
                jprev = jg - nbuf

                @pl.when(jprev >= 0)
                def _():
                    out_desc(lax.max(jprev, 0), b2).wait()

                @pl.when(jg < nchunk)
                def _():
                    gather_desc(lax.min(jg, nchunk - 1), b2).start()

            return carry

        lax.fori_loop(0, n_outer, body, 0)

        for i in range(nbuf - h):
            c = nchunk - (nbuf - h) + i
            out_desc(c, c % nbuf).wait()

    return k(idx3, weight)


def kernel(token_ids, weight):
    b, s = token_ids.shape
    v, d = weight.shape
    n_total = b * s
    info = plsc.get_sparse_core_info()
    nc, ns = info.num_cores, info.num_subcores
    nw = nc * ns
    g = 128
    nbuf, h = 8, 4
    idx3 = token_ids.reshape(nw, n_total // (nw * g), g)
    out = _gather_call(n_total, d, nw, nc, g, nbuf, h, idx3, weight)
    return out.reshape(b, s, d)


# speedup vs baseline: 1.2435x; 1.1121x over previous
import functools

import jax
import jax.numpy as jnp
from jax import lax
from jax.experimental import pallas as pl
from jax.experimental.pallas import tpu as pltpu
from jax.experimental.pallas import tpu_sc as plsc


def _gather_call(seq, bt, d, nw, nc, idx4, wtp):
    g = bt // nw
    nbuf = 4
    half = 2
    mesh = plsc.VectorSubcoreMesh(core_axis_name="c", subcore_axis_name="s")
    scratch = [
        pltpu.VMEM((seq, g), jnp.int32),
        pltpu.VMEM((nbuf, g, 2 * d), jnp.float32),
        pltpu.VMEM((half, g, d), jnp.float32),
    ] + [pltpu.SemaphoreType.DMA] * (nbuf + half)

    @functools.partial(
        pl.kernel,
        mesh=mesh,
        out_type=jax.ShapeDtypeStruct((bt, seq, d), jnp.float32),
        compiler_params=pltpu.CompilerParams(use_tc_tiling_on_sc=True),
        scratch_types=scratch,
    )
    def k(idx_hbm, tab_hbm, out_hbm, idx_v, g_v, o_v, *sems):
        gsem = sems[:nbuf]
        osem = sems[nbuf:]
        wid = lax.axis_index("s") * nc + lax.axis_index("c")
        pltpu.sync_copy(idx_hbm.at[wid], idx_v)
        row0 = wid * g

        def gather_desc(si, slot):
            return pltpu.make_async_copy(
                tab_hbm.at[idx_v.at[si]], g_v.at[slot], gsem[slot]
            )

        def out_desc(si, oslot):
            return pltpu.make_async_copy(
                o_v.at[oslot],
                out_hbm.at[pl.ds(row0, g), si, :],
                osem[oslot],
            )

        def compact(slot, oslot):
            gref = g_v.at[slot]
            oref = o_v.at[oslot]

            def body(i, carry):
                for u in range(d // 16):
                    oref[i, pl.ds(16 * u, 16)] = gref[i, pl.ds(16 * u, 16)]
                return carry

            lax.fori_loop(0, g, body, 0)

        for si in range(nbuf):
            gather_desc(si, si).start()

        def blk(bi, carry):
            for bsl in range(nbuf):
                si = bi * nbuf + bsl
                oslot = bsl % half
                gather_desc(si, bsl).wait()

                @pl.when(si >= half)
                def _():
                    out_desc(lax.max(si - half, 0), oslot).wait()

                compact(bsl, oslot)
                out_desc(si, oslot).start()

                @pl.when(si + nbuf < seq)
                def _():
                    gather_desc(lax.min(si + nbuf, seq - 1), bsl).start()

            return carry

        lax.fori_loop(0, seq // nbuf, blk, 0)
        out_desc(seq - 2, 0).wait()
        out_desc(seq - 1, 1).wait()

    return k(idx4, wtp)


def kernel(token_ids, weight):
    bt, seq = token_ids.shape
    v, d = weight.shape
    info = plsc.get_sparse_core_info()
    nc, ns = info.num_cores, info.num_subcores
    nw = nc * ns
    g = bt // nw
    wtp = jnp.pad(weight, ((0, 0), (0, d)))
    idx4 = token_ids.reshape(nw, g, seq).transpose(0, 2, 1)
    out = _gather_call(seq, bt, d, nw, nc, idx4, wtp)
    return out
